# Initial kernel scaffold; baseline (speedup 1.0000x reference)
import jax
import jax.numpy as jnp
from jax.experimental import pallas as pl


def kernel(feature, edge_index, W1a, b1a, W1b, b1b, W2a, b2a, W2b, b2b):
    raise NotImplementedError("write your pallas kernel here")


# speedup vs baseline: 2.8707x; 2.8707x over previous
import functools

import jax
import jax.numpy as jnp
from jax import lax
from jax.experimental import pallas as pl
from jax.experimental.pallas import tpu as pltpu
from jax.experimental.pallas import tpu_sc as plsc

N = 10000
D = 256
H = 512
E = 160000

NC = 2
NS = 16
LANES = 16
EB = 80
NPAD = 10240
NODES_PER_TILE = NPAD // NS
ZROWS = 40
EDGES_PER_TILE = E // NS
NBATCH = EDGES_PER_TILE // EB


def _make_seg_sum(nblocks, row_stride, base_mult):
    P = nblocks // NC
    mesh = plsc.VectorSubcoreMesh(core_axis_name="c", subcore_axis_name="s",
                                  num_cores=NC, num_subcores=NS)

    @functools.partial(
        pl.kernel, mesh=mesh,
        out_type=jax.ShapeDtypeStruct((nblocks * NPAD, 128), jnp.float32),
        scratch_types=[
            pltpu.VMEM((EB,), jnp.int32),
            pltpu.VMEM((EB,), jnp.int32),
            pltpu.VMEM((EB, 128), jnp.float32),
            pltpu.VMEM((ZROWS, 128), jnp.float32),
            pltpu.VMEM_SHARED((NPAD, 128), jnp.float32),
            pltpu.SemaphoreType.DMA,
        ],
    )
    def seg(x_hbm, src_hbm, dst_hbm, out_hbm, gidx, didx, rows, zrow, acc, sem):
        core = lax.axis_index("c")
        tid = lax.axis_index("s")
        r0 = tid * NODES_PER_TILE
        e0 = tid * EDGES_PER_TILE
        zero16 = jnp.zeros((LANES,), jnp.float32)
        for i in range(ZROWS):
            for k in range(128 // LANES):
                zrow[i, pl.ds(k * LANES, LANES)] = zero16
        for p in range(P):
            cblk = core * P + p
            base = cblk * base_mult
            for u in range(NODES_PER_TILE // ZROWS):
                pltpu.sync_copy(zrow, acc.at[pl.ds(r0 + u * ZROWS, ZROWS)])
            plsc.subcore_barrier()

            def body(it, carry):
                off = e0 + it * EB
                pltpu.sync_copy(src_hbm.at[pl.ds(off, EB)], gidx)
                pltpu.sync_copy(dst_hbm.at[pl.ds(off, EB)], didx)
                for j in range(EB // LANES):
                    s = gidx[pl.ds(j * LANES, LANES)]
                    gidx[pl.ds(j * LANES, LANES)] = s * row_stride + base
                pltpu.async_copy(x_hbm.at[gidx], rows, sem).wait()
                pltpu.sync_copy(rows, acc.at[didx], add=True)
                return carry

            lax.fori_loop(0, NBATCH, body, 0)
            plsc.subcore_barrier()
            pltpu.sync_copy(
                acc.at[pl.ds(r0, NODES_PER_TILE)],
                out_hbm.at[pl.ds(cblk * NPAD + r0, NODES_PER_TILE)])

    return seg


_seg1 = _make_seg_sum(nblocks=2, row_stride=2, base_mult=1)
_seg2 = _make_seg_sum(nblocks=8, row_stride=1, base_mult=N)

_RB = 1000


def _mlp1(f, g1, w1a, b1a, w1b, b1b):
    def body(f_ref, g_ref, wa_ref, ba_ref, wb_ref, bb_ref, out_ref):
        agg = jnp.concatenate([g_ref[0], g_ref[1]], axis=1)
        t = f_ref[...] + agg
        v = jnp.where(t > 0.0, t, jnp.exp(t) - 1.0)
        h = jnp.maximum(
            jnp.dot(v, wa_ref[...], preferred_element_type=jnp.float32)
            + ba_ref[...], 0.0)
        x3 = jnp.dot(h, wb_ref[...], preferred_element_type=jnp.float32) \
            + bb_ref[...]
        for j in range(8):
            out_ref[j] = x3[:, j * 128:(j + 1) * 128]

    return pl.pallas_call(
        body,
        grid=(N // _RB,),
        in_specs=[
            pl.BlockSpec((_RB, D), lambda i: (i, 0)),
            pl.BlockSpec((2, _RB, 128), lambda i: (0, i, 0)),
            pl.BlockSpec((D, H), lambda i: (0, 0)),
            pl.BlockSpec((1, H), lambda i: (0, 0)),
            pl.BlockSpec((H, 2 * H), lambda i: (0, 0)),
            pl.BlockSpec((1, 2 * H), lambda i: (0, 0)),
        ],
        out_specs=pl.BlockSpec((8, _RB, 128), lambda i: (0, i, 0)),
        out_shape=jax.ShapeDtypeStruct((8, N, 128), jnp.float32),
    )(f, g1, w1a, b1a, w1b, b1b)


def _mlp2(x3, g2, w2a, b2a, w2b, b2b):
    def body(x_ref, g_ref, wa_ref, ba_ref, wb_ref, bb_ref, out_ref):
        z = jnp.concatenate([x_ref[j] + g_ref[j] for j in range(8)], axis=1)
        y = jnp.maximum(
            jnp.dot(z, wa_ref[...], preferred_element_type=jnp.float32)
            + ba_ref[...], 0.0)
        out_ref[...] = jnp.dot(
            y, wb_ref[...], preferred_element_type=jnp.float32) + bb_ref[...]

    return pl.pallas_call(
        body,
        grid=(N // _RB,),
        in_specs=[
            pl.BlockSpec((8, _RB, 128), lambda i: (0, i, 0)),
            pl.BlockSpec((8, _RB, 128), lambda i: (0, i, 0)),
            pl.BlockSpec((2 * H, H), lambda i: (0, 0)),
            pl.BlockSpec((1, H), lambda i: (0, 0)),
            pl.BlockSpec((H, D), lambda i: (0, 0)),
            pl.BlockSpec((1, D), lambda i: (0, 0)),
        ],
        out_specs=pl.BlockSpec((_RB, D), lambda i: (i, 0)),
        out_shape=jax.ShapeDtypeStruct((N, D), jnp.float32),
    )(x3, g2, w2a, b2a, w2b, b2b)


def kernel(feature, edge_index, W1a, b1a, W1b, b1b, W2a, b2a, W2b, b2b):
    src = edge_index[0]
    dst = edge_index[1]
    w1a_eff = W1a[:D] + W1a[D:]
    f2 = feature.reshape(2 * N, 128)
    g1 = _seg1(f2, src, dst).reshape(2, NPAD, 128)
    x3 = _mlp1(feature, g1, w1a_eff, b1a.reshape(1, H),
               W1b, b1b.reshape(1, 2 * H))
    g2 = _seg2(x3.reshape(8 * N, 128), src, dst).reshape(8, NPAD, 128)
    return _mlp2(x3, g2, W2a, b2a.reshape(1, H),
                 W2b, b2b.reshape(1, D))
